# baseline (speedup 1.0000x reference)
import functools

import jax
import jax.numpy as jnp
from jax import lax
from jax.experimental import pallas as pl
from jax.experimental.pallas import tpu as pltpu

_BB = 4
_T = 64
_H = 4


def _bf(x):
    return x.astype(jnp.bfloat16)


def _dot(a, b):
    return jnp.dot(a, b, preferred_element_type=jnp.float32)


def _leaky(x):
    return jnp.maximum(x, 0.2 * x)


def _row_repeat(s, f):
    n = s.shape[0]
    return jnp.broadcast_to(s[:, None, :], (n, f, n)).reshape(n * f, n)


def _ln_multi(y, gamma, beta, eps=1e-5):
    n_el = float(y.shape[1] * y.shape[2])
    s1 = jnp.sum(jnp.sum(y, axis=2, keepdims=True), axis=1, keepdims=True)
    s2 = jnp.sum(jnp.sum(y * y, axis=2, keepdims=True), axis=1, keepdims=True)
    mean = s1 / n_el
    var = s2 / n_el - mean * mean
    return (y - mean) * lax.rsqrt(var + eps) * gamma[None] + beta[None]


def _gc_multi(ys, wtl_exp, wtr, wblk, wsp, fmask, red_gblk, bias_full,
              gblkt_q, mfo_q, red_from_a1):
    nb, tl, d1 = ys.shape
    d2 = wblk.shape[1]
    n = red_gblk.shape[1]
    fr = fmask.shape[0] // (2 * _H)
    dr = fmask.shape[1]
    fexp = d2 // n
    ys_bf = _bf(ys)
    ys_flat_bf = ys_bf.reshape(nb * tl, d1)

    plx = _dot(ys_flat_bf, wtl_exp)
    projr = [lax.dot_general(wtr, ys_bf[g], (((1,), (1,)), ((), ())),
                             preferred_element_type=jnp.float32)
             for g in range(nb)]
    sp = None
    for h in range(_H):
        plh = plx[:, h * tl:(h + 1) * tl]
        prh = jnp.concatenate(
            [jnp.broadcast_to(projr[g][h:h + 1], (tl, tl))
             for g in range(nb)], axis=0)
        pmx = jnp.concatenate(
            [jnp.broadcast_to(jnp.max(projr[g][h:h + 1], axis=1,
                                      keepdims=True), (tl, tl))
             for g in range(nb)], axis=0)
        eh = _leaky(plh + prh) - _leaky(plh + pmx)
        ph = jnp.exp(eh)
        ph = ph / (jnp.sum(ph, axis=-1, keepdims=True) * float(_H))
        sp = ph if sp is None else sp + ph

    a1b = [_bf(_dot(_bf(sp[g * tl:(g + 1) * tl]), ys_bf[g]))
           for g in range(nb)]
    y2_flat = _dot(jnp.concatenate(a1b, axis=0), wblk)
    y2 = y2_flat.reshape(nb, tl, d2)
    y2b = _bf(y2)

    red_src = a1b if red_from_a1 else [y2b[g] for g in range(nb)]
    ss = [jnp.sum((_dot(wsp, src) * fmask).reshape(2 * _H, fr, dr), axis=1)
          for src in red_src]
    proj_s = _dot(_bf(jnp.concatenate(ss, axis=0)), red_gblk)
    ps3 = proj_s.reshape(nb, 2 * _H, n)
    r_, l_ = ps3[:, _H:], ps3[:, :_H]
    etmax = _leaky(l_ + jnp.max(r_, axis=-1, keepdims=True))
    et = _leaky(r_[:, :, :, None] + l_[:, :, None, :]) - etmax[:, :, None, :]
    pt = jnp.exp(et)
    pt = pt / jnp.sum(pt, axis=2, keepdims=True)
    spt = jnp.sum(pt, axis=1)

    outs = []
    for g in range(nb):
        q = _dot(_bf(_row_repeat(spt[g], fexp)), gblkt_q) * mfo_q
        outs.append(_dot(y2b[g], _bf(q)))
    out = jnp.concatenate(outs, axis=0).reshape(nb, tl, d2)
    return out + bias_full[None]


def _chain_multi(x, ps):
    l_s, l_r = ps[0:9], ps[9:18]
    lng, lnb = ps[18], ps[19]
    l_e = ps[20:29]
    y = _gc_multi(x, *l_s, True)
    y = jnp.concatenate([y, y], axis=1)
    z = _gc_multi(y, *l_r, False)
    z = jnp.tanh(_ln_multi(z, lng, lnb))
    y = z + y
    y = y[:, :_T, :]
    y = _gc_multi(y, *l_e, False)
    return y + x


def _conv3_multi(x_bf, w0, w1, w2, b):
    nb, tl, c = x_bf.shape
    xm = jnp.concatenate([x_bf[:, :1], x_bf[:, :-1]], axis=1)
    xp = jnp.concatenate([x_bf[:, 1:], x_bf[:, -1:]], axis=1)
    r = (_dot(xm.reshape(nb * tl, c), w0)
         + _dot(x_bf.reshape(nb * tl, c), w1)
         + _dot(xp.reshape(nb * tl, c), w2) + b)
    return r.reshape(nb, tl, r.shape[1])


def _gaze_multi(x, gz):
    (w10, w11, w12, b1, g1, be1,
     w20, w21, w22, b2, g2, be2,
     w30, w31, w32, b3, g3, be3,
     w40, w41, w42, b4) = gz
    y = jnp.tanh(_ln_multi(_conv3_multi(_bf(x), w10, w11, w12, b1), g1, be1))
    y = jnp.tanh(_ln_multi(_conv3_multi(_bf(y), w20, w21, w22, b2), g2, be2))
    y = jnp.tanh(_ln_multi(_conv3_multi(_bf(y), w30, w31, w32, b3), g3, be3))
    y = jnp.tanh(_conv3_multi(_bf(y), w40, w41, w42, b4))
    return y


def _net_kernel(*refs):
    pose_ref, gaze_ref = refs[0], refs[1]
    consts = [r[...] for r in refs[2:-1]]
    o_ref = refs[-1]
    gz = consts[0:22]
    po = consts[22:51]
    mn = consts[51:80]
    xp = pose_ref[...].astype(jnp.float32)
    xg = gaze_ref[...].astype(jnp.float32)
    gout = _gaze_multi(xg, gz)
    pout = _chain_multi(xp, po)
    xm = jnp.concatenate([pout, gout], axis=2)
    o_ref[...] = _chain_multi(xm, mn).astype(o_ref.dtype)


def _gblk(node_n, f):
    return jnp.kron(jnp.eye(node_n, dtype=jnp.float32),
                    jnp.ones((f, 1), jnp.float32))


def _prep_layer(att_t, fw, att_s, bias, node_n, start):
    fin, fout = fw.shape
    at = att_t[:, :, 0]
    d1 = at.shape[1] // 2
    wblk = _bf(jnp.kron(jnp.eye(node_n, dtype=fw.dtype), fw))
    asp = att_s[:, :, 0]
    tl = asp.shape[1] // (2 * fout)
    wtl_exp = _bf(jnp.repeat(jnp.transpose(at[:, :d1]), tl, axis=1))
    wtr = _bf(at[:, d1:])
    wsl = asp[:, :tl * fout].reshape(_H, tl, fout)
    wsr = asp[:, tl * fout:].reshape(_H, tl, fout)
    ws = jnp.concatenate([wsl, wsr], axis=0)
    if start:
        wsr_ = jnp.einsum("htf,gf->htg", ws, fw)
        frr = fin
    else:
        wsr_ = ws
        frr = fout
    wsp = _bf(jnp.transpose(wsr_, (0, 2, 1)).reshape(2 * _H * frr, tl))
    fmask = jnp.tile(jnp.eye(frr, dtype=jnp.float32), (2 * _H, node_n))
    red_gblk = _bf(_gblk(node_n, frr))
    bias_full = jnp.broadcast_to(bias[:, None].astype(jnp.float32),
                                 (tl, node_n * fout)) + jnp.zeros(
                                     (tl, node_n * fout), jnp.float32)
    gblkt_q = _bf(jnp.transpose(_gblk(node_n, fout)))
    mfo_q = jnp.tile(jnp.eye(fout, dtype=jnp.float32),
                     (node_n, node_n)) * (1.0 / _H)
    return [wtl_exp, wtr, wblk, wsp, fmask, red_gblk, bias_full, gblkt_q,
            mfo_q]


def _taps(w, b):
    return [_bf(jnp.transpose(w[:, :, 0])), _bf(jnp.transpose(w[:, :, 1])),
            _bf(jnp.transpose(w[:, :, 2])), b[None, :].astype(jnp.float32)]


def kernel(x,
           gz_w1, gz_b1, gz_ln1_g, gz_ln1_b,
           gz_w2, gz_b2, gz_ln2_g, gz_ln2_b,
           gz_w3, gz_b3, gz_ln3_g, gz_ln3_b,
           gz_w4, gz_b4,
           po_start_att_t, po_start_fw, po_start_att_s, po_start_bias,
           po_res0_att_t, po_res0_fw, po_res0_att_s, po_res0_bias,
           po_res0_lng, po_res0_lnb,
           po_end_att_t, po_end_fw, po_end_att_s, po_end_bias,
           mn_start_att_t, mn_start_fw, mn_start_att_s, mn_start_bias,
           mn_res0_att_t, mn_res0_fw, mn_res0_att_s, mn_res0_bias,
           mn_res0_lng, mn_res0_lnb,
           mn_end_att_t, mn_end_fw, mn_end_att_s, mn_end_bias):
    B, fin, node_n, T = x.shape
    npo = node_n - 1
    pose_l = jnp.transpose(x[:, :, :-1, :], (0, 3, 2, 1)).reshape(B, T, npo * fin)
    gaze_l = jnp.transpose(x[:, :, -1, :], (0, 2, 1))

    gz = (_taps(gz_w1, gz_b1) + [jnp.transpose(gz_ln1_g), jnp.transpose(gz_ln1_b)]
          + _taps(gz_w2, gz_b2) + [jnp.transpose(gz_ln2_g), jnp.transpose(gz_ln2_b)]
          + _taps(gz_w3, gz_b3) + [jnp.transpose(gz_ln3_g), jnp.transpose(gz_ln3_b)]
          + _taps(gz_w4, gz_b4))

    def chain_consts(s_att_t, s_fw, s_att_s, s_b, r_att_t, r_fw, r_att_s, r_b,
                     lng, lnb, e_att_t, e_fw, e_att_s, e_b, n):
        latent = s_fw.shape[1]
        dl = n * latent
        cs = _prep_layer(s_att_t, s_fw, s_att_s, s_b, n, True)
        cs += _prep_layer(r_att_t, r_fw, r_att_s, r_b, n, False)
        cs += [jnp.transpose(lng, (2, 1, 0)).reshape(2 * _T, dl),
               jnp.transpose(lnb, (2, 1, 0)).reshape(2 * _T, dl)]
        cs += _prep_layer(e_att_t, e_fw, e_att_s, e_b, n, False)
        return cs

    po = chain_consts(po_start_att_t, po_start_fw, po_start_att_s, po_start_bias,
                      po_res0_att_t, po_res0_fw, po_res0_att_s, po_res0_bias,
                      po_res0_lng, po_res0_lnb,
                      po_end_att_t, po_end_fw, po_end_att_s, po_end_bias, npo)
    mn = chain_consts(mn_start_att_t, mn_start_fw, mn_start_att_s, mn_start_bias,
                      mn_res0_att_t, mn_res0_fw, mn_res0_att_s, mn_res0_bias,
                      mn_res0_lng, mn_res0_lnb,
                      mn_end_att_t, mn_end_fw, mn_end_att_s, mn_end_bias, node_n)

    consts = gz + po + mn

    def _const_spec(a):
        if a.ndim == 3:
            return pl.BlockSpec(a.shape, lambda b: (0, 0, 0))
        return pl.BlockSpec(a.shape, lambda b: (0, 0))

    out = pl.pallas_call(
        _net_kernel,
        out_shape=jax.ShapeDtypeStruct((B, T, node_n * fin), jnp.float32),
        grid=(B // _BB,),
        in_specs=[pl.BlockSpec((_BB, T, npo * fin), lambda b: (b, 0, 0)),
                  pl.BlockSpec((_BB, T, fin), lambda b: (b, 0, 0))]
                 + [_const_spec(a) for a in consts],
        out_specs=pl.BlockSpec((_BB, T, node_n * fin), lambda b: (b, 0, 0)),
        compiler_params=pltpu.CompilerParams(dimension_semantics=("parallel",)),
    )(pose_l, gaze_l, *consts)

    return jnp.transpose(out.reshape(B, T, node_n, fin), (0, 3, 2, 1))


# speedup vs baseline: 1.0714x; 1.0714x over previous
import functools

import jax
import jax.numpy as jnp
from jax import lax
from jax.experimental import pallas as pl
from jax.experimental.pallas import tpu as pltpu

_BB = 4
_T = 64
_H = 4


def _bf(x):
    return x.astype(jnp.bfloat16)


def _dot(a, b):
    return jnp.dot(a, b, preferred_element_type=jnp.float32)


def _leaky(x):
    return jnp.maximum(x, 0.2 * x)


def _row_repeat(s, f):
    n = s.shape[0]
    return jnp.broadcast_to(s[:, None, :], (n, f, n)).reshape(n * f, n)


def _ln_multi(y, gamma, beta, eps=1e-5):
    n_el = float(y.shape[1] * y.shape[2])
    s1 = jnp.sum(jnp.sum(y, axis=2, keepdims=True), axis=1, keepdims=True)
    s2 = jnp.sum(jnp.sum(y * y, axis=2, keepdims=True), axis=1, keepdims=True)
    mean = s1 / n_el
    var = s2 / n_el - mean * mean
    return (y - mean) * lax.rsqrt(var + eps) * gamma[None] + beta[None]


def _gc_multi(ys, wt, wblk, wsp, fmask, red_gblk, bias_full, gblkt_q, mfo_q,
              red_from_a1):
    nb, tl, d1 = ys.shape
    d2 = wblk.shape[1]
    n = red_gblk.shape[1]
    fr = fmask.shape[0] // (2 * _H)
    dr = fmask.shape[1]
    fexp = d2 // n
    inv_h = 1.0 / _H
    ys_bf = _bf(ys)

    projs = [lax.dot_general(wt, ys_bf[g], (((1,), (1,)), ((), ())),
                             preferred_element_type=jnp.float32)
             for g in range(nb)]
    proj = jnp.concatenate(projs, axis=0).reshape(nb, 2 * _H, tl)
    pl_, pr_ = proj[:, :_H], proj[:, _H:]
    emax = _leaky(pl_ + jnp.max(pr_, axis=-1, keepdims=True))
    e = _leaky(pl_[:, :, :, None] + pr_[:, :, None, :]) - emax[:, :, :, None]
    p = jnp.exp(e)
    p = p / (jnp.sum(p, axis=-1, keepdims=True) * float(_H))
    sp = jnp.sum(p, axis=1)

    a1b = [_bf(_dot(_bf(sp[g]), ys_bf[g])) for g in range(nb)]
    y2_flat = _dot(jnp.concatenate(a1b, axis=0), wblk)
    y2 = y2_flat.reshape(nb, tl, d2)
    y2b = _bf(y2)

    red_src = a1b if red_from_a1 else [y2b[g] for g in range(nb)]
    ss = [jnp.sum((_dot(wsp, src) * fmask).reshape(2 * _H, fr, dr), axis=1)
          for src in red_src]
    proj_s = _dot(_bf(jnp.concatenate(ss, axis=0)), red_gblk)
    ps3 = proj_s.reshape(nb, 2 * _H, n)
    r_, l_ = ps3[:, _H:], ps3[:, :_H]
    etmax = _leaky(l_ + jnp.max(r_, axis=-1, keepdims=True))
    et = _leaky(r_[:, :, :, None] + l_[:, :, None, :]) - etmax[:, :, None, :]
    pt = jnp.exp(et)
    pt = pt / jnp.sum(pt, axis=2, keepdims=True)
    spt = jnp.sum(pt, axis=1)

    outs = []
    for g in range(nb):
        q = _dot(_bf(_row_repeat(spt[g], fexp)), gblkt_q) * mfo_q
        outs.append(_dot(y2b[g], _bf(q)))
    out = jnp.concatenate(outs, axis=0).reshape(nb, tl, d2)
    return out + bias_full[None]


def _chain_multi(x, ps):
    l_s, l_r = ps[0:8], ps[8:16]
    lng, lnb = ps[16], ps[17]
    l_e = ps[18:26]
    y = _gc_multi(x, *l_s, True)
    y = jnp.concatenate([y, y], axis=1)
    z = _gc_multi(y, *l_r, False)
    z = jnp.tanh(_ln_multi(z, lng, lnb))
    y = z + y
    y = y[:, :_T, :]
    y = _gc_multi(y, *l_e, False)
    return y + x


def _conv3_multi(x_bf, w0, w1, w2, b):
    nb, tl, c = x_bf.shape
    xm = jnp.concatenate([x_bf[:, :1], x_bf[:, :-1]], axis=1)
    xp = jnp.concatenate([x_bf[:, 1:], x_bf[:, -1:]], axis=1)
    r = (_dot(xm.reshape(nb * tl, c), w0)
         + _dot(x_bf.reshape(nb * tl, c), w1)
         + _dot(xp.reshape(nb * tl, c), w2) + b)
    return r.reshape(nb, tl, r.shape[1])


def _gaze_multi(x, gz):
    (w10, w11, w12, b1, g1, be1,
     w20, w21, w22, b2, g2, be2,
     w30, w31, w32, b3, g3, be3,
     w40, w41, w42, b4) = gz
    y = jnp.tanh(_ln_multi(_conv3_multi(_bf(x), w10, w11, w12, b1), g1, be1))
    y = jnp.tanh(_ln_multi(_conv3_multi(_bf(y), w20, w21, w22, b2), g2, be2))
    y = jnp.tanh(_ln_multi(_conv3_multi(_bf(y), w30, w31, w32, b3), g3, be3))
    y = jnp.tanh(_conv3_multi(_bf(y), w40, w41, w42, b4))
    return y


def _net_kernel(*refs):
    pose_ref, gaze_ref = refs[0], refs[1]
    consts = [r[...] for r in refs[2:-1]]
    o_ref = refs[-1]
    gz = consts[0:22]
    po = consts[22:48]
    mn = consts[48:74]
    xp = pose_ref[...].astype(jnp.float32)
    xg = gaze_ref[...].astype(jnp.float32)
    gout = _gaze_multi(xg, gz)
    pout = _chain_multi(xp, po)
    xm = jnp.concatenate([pout, gout], axis=2)
    o_ref[...] = _chain_multi(xm, mn).astype(o_ref.dtype)


def _gblk(node_n, f):
    return jnp.kron(jnp.eye(node_n, dtype=jnp.float32),
                    jnp.ones((f, 1), jnp.float32))


def _prep_layer(att_t, fw, att_s, bias, node_n, start):
    fin, fout = fw.shape
    at = att_t[:, :, 0]
    d1 = at.shape[1] // 2
    wt = _bf(jnp.concatenate([at[:, :d1], at[:, d1:]], axis=0))
    wblk = _bf(jnp.kron(jnp.eye(node_n, dtype=fw.dtype), fw))
    asp = att_s[:, :, 0]
    tl = asp.shape[1] // (2 * fout)
    wsl = asp[:, :tl * fout].reshape(_H, tl, fout)
    wsr = asp[:, tl * fout:].reshape(_H, tl, fout)
    ws = jnp.concatenate([wsl, wsr], axis=0)
    if start:
        wsr_ = jnp.einsum("htf,gf->htg", ws, fw)
        frr = fin
    else:
        wsr_ = ws
        frr = fout
    wsp = _bf(jnp.transpose(wsr_, (0, 2, 1)).reshape(2 * _H * frr, tl))
    fmask = jnp.tile(jnp.eye(frr, dtype=jnp.float32), (2 * _H, node_n))
    red_gblk = _bf(_gblk(node_n, frr))
    bias_full = jnp.broadcast_to(bias[:, None].astype(jnp.float32),
                                 (tl, node_n * fout)) + jnp.zeros(
                                     (tl, node_n * fout), jnp.float32)
    gblkt_q = _bf(jnp.transpose(_gblk(node_n, fout)))
    mfo_q = jnp.tile(jnp.eye(fout, dtype=jnp.float32),
                     (node_n, node_n)) * (1.0 / _H)
    return [wt, wblk, wsp, fmask, red_gblk, bias_full, gblkt_q, mfo_q]


def _taps(w, b):
    return [_bf(jnp.transpose(w[:, :, 0])), _bf(jnp.transpose(w[:, :, 1])),
            _bf(jnp.transpose(w[:, :, 2])), b[None, :].astype(jnp.float32)]


def kernel(x,
           gz_w1, gz_b1, gz_ln1_g, gz_ln1_b,
           gz_w2, gz_b2, gz_ln2_g, gz_ln2_b,
           gz_w3, gz_b3, gz_ln3_g, gz_ln3_b,
           gz_w4, gz_b4,
           po_start_att_t, po_start_fw, po_start_att_s, po_start_bias,
           po_res0_att_t, po_res0_fw, po_res0_att_s, po_res0_bias,
           po_res0_lng, po_res0_lnb,
           po_end_att_t, po_end_fw, po_end_att_s, po_end_bias,
           mn_start_att_t, mn_start_fw, mn_start_att_s, mn_start_bias,
           mn_res0_att_t, mn_res0_fw, mn_res0_att_s, mn_res0_bias,
           mn_res0_lng, mn_res0_lnb,
           mn_end_att_t, mn_end_fw, mn_end_att_s, mn_end_bias):
    B, fin, node_n, T = x.shape
    npo = node_n - 1
    pose_l = jnp.transpose(x[:, :, :-1, :], (0, 3, 2, 1)).reshape(B, T, npo * fin)
    gaze_l = jnp.transpose(x[:, :, -1, :], (0, 2, 1))

    gz = (_taps(gz_w1, gz_b1) + [jnp.transpose(gz_ln1_g), jnp.transpose(gz_ln1_b)]
          + _taps(gz_w2, gz_b2) + [jnp.transpose(gz_ln2_g), jnp.transpose(gz_ln2_b)]
          + _taps(gz_w3, gz_b3) + [jnp.transpose(gz_ln3_g), jnp.transpose(gz_ln3_b)]
          + _taps(gz_w4, gz_b4))

    def chain_consts(s_att_t, s_fw, s_att_s, s_b, r_att_t, r_fw, r_att_s, r_b,
                     lng, lnb, e_att_t, e_fw, e_att_s, e_b, n):
        latent = s_fw.shape[1]
        dl = n * latent
        cs = _prep_layer(s_att_t, s_fw, s_att_s, s_b, n, True)
        cs += _prep_layer(r_att_t, r_fw, r_att_s, r_b, n, False)
        cs += [jnp.transpose(lng, (2, 1, 0)).reshape(2 * _T, dl),
               jnp.transpose(lnb, (2, 1, 0)).reshape(2 * _T, dl)]
        cs += _prep_layer(e_att_t, e_fw, e_att_s, e_b, n, False)
        return cs

    po = chain_consts(po_start_att_t, po_start_fw, po_start_att_s, po_start_bias,
                      po_res0_att_t, po_res0_fw, po_res0_att_s, po_res0_bias,
                      po_res0_lng, po_res0_lnb,
                      po_end_att_t, po_end_fw, po_end_att_s, po_end_bias, npo)
    mn = chain_consts(mn_start_att_t, mn_start_fw, mn_start_att_s, mn_start_bias,
                      mn_res0_att_t, mn_res0_fw, mn_res0_att_s, mn_res0_bias,
                      mn_res0_lng, mn_res0_lnb,
                      mn_end_att_t, mn_end_fw, mn_end_att_s, mn_end_bias, node_n)

    consts = gz + po + mn

    def _const_spec(a):
        if a.ndim == 3:
            return pl.BlockSpec(a.shape, lambda b: (0, 0, 0))
        return pl.BlockSpec(a.shape, lambda b: (0, 0))

    out = pl.pallas_call(
        _net_kernel,
        out_shape=jax.ShapeDtypeStruct((B, T, node_n * fin), jnp.float32),
        grid=(B // _BB,),
        in_specs=[pl.BlockSpec((_BB, T, npo * fin), lambda b: (b, 0, 0)),
                  pl.BlockSpec((_BB, T, fin), lambda b: (b, 0, 0))]
                 + [_const_spec(a) for a in consts],
        out_specs=pl.BlockSpec((_BB, T, node_n * fin), lambda b: (b, 0, 0)),
        compiler_params=pltpu.CompilerParams(dimension_semantics=("parallel",)),
    )(pose_l, gaze_l, *consts)

    return jnp.transpose(out.reshape(B, T, node_n, fin), (0, 3, 2, 1))


# speedup vs baseline: 1.1876x; 1.1085x over previous
import functools

import jax
import jax.numpy as jnp
from jax import lax
from jax.experimental import pallas as pl
from jax.experimental.pallas import tpu as pltpu

_BB = 4
_T = 64
_H = 4


def _bf(x):
    return x.astype(jnp.bfloat16)


def _dot(a, b):
    return jnp.dot(a, b, preferred_element_type=jnp.float32)


def _leaky(x):
    return jnp.maximum(x, 0.2 * x)


def _row_repeat(s, f):
    n = s.shape[0]
    return jnp.broadcast_to(s[:, None, :], (n, f, n)).reshape(n * f, n)


def _ln_multi(y, gamma, beta, eps=1e-5):
    n_el = float(y.shape[1] * y.shape[2])
    s1 = jnp.sum(jnp.sum(y, axis=2, keepdims=True), axis=1, keepdims=True)
    s2 = jnp.sum(jnp.sum(y * y, axis=2, keepdims=True), axis=1, keepdims=True)
    mean = s1 / n_el
    var = s2 / n_el - mean * mean
    return (y - mean) * lax.rsqrt(var + eps) * gamma[None] + beta[None]


def _gc_multi(ys, wt, wblk, wsp, fmask, red_gblk, bias_full, gblkt_q, mfo_q,
              red_from_a1):
    nb, tl, d1 = ys.shape
    d2 = wblk.shape[1]
    n = red_gblk.shape[1]
    fr = fmask.shape[0] // (2 * _H)
    dr = fmask.shape[1]
    fexp = d2 // n
    inv_h = 1.0 / _H
    ys_bf = _bf(ys)

    projs = [lax.dot_general(wt, ys_bf[g], (((1,), (1,)), ((), ())),
                             preferred_element_type=jnp.float32)
             for g in range(nb)]
    proj = jnp.concatenate(projs, axis=0).reshape(nb, 2 * _H, tl)
    pl_, pr_ = proj[:, :_H], proj[:, _H:]
    emax = _leaky(pl_ + jnp.max(pr_, axis=-1, keepdims=True))
    e = _leaky(pr_[:, :, :, None] + pl_[:, :, None, :]) - emax[:, :, None, :]
    p = jnp.exp(e)
    p = p / (jnp.sum(p, axis=2, keepdims=True) * float(_H))
    sp = jnp.sum(p, axis=1)

    a1b = [_bf(lax.dot_general(_bf(sp[g]), ys_bf[g], (((0,), (0,)), ((), ())),
                               preferred_element_type=jnp.float32))
           for g in range(nb)]
    y2_flat = _dot(jnp.concatenate(a1b, axis=0), wblk)
    y2 = y2_flat.reshape(nb, tl, d2)
    y2b = _bf(y2)

    red_src = a1b if red_from_a1 else [y2b[g] for g in range(nb)]
    ss = [jnp.sum((_dot(wsp, src) * fmask).reshape(2 * _H, fr, dr), axis=1)
          for src in red_src]
    proj_s = _dot(_bf(jnp.concatenate(ss, axis=0)), red_gblk)
    ps3 = proj_s.reshape(nb, 2 * _H, n)
    r_, l_ = ps3[:, _H:], ps3[:, :_H]
    etmax = _leaky(l_ + jnp.max(r_, axis=-1, keepdims=True))
    et = _leaky(r_[:, :, :, None] + l_[:, :, None, :]) - etmax[:, :, None, :]
    pt = jnp.exp(et)
    pt = pt / jnp.sum(pt, axis=2, keepdims=True)
    spt = jnp.sum(pt, axis=1)

    outs = []
    for g in range(nb):
        q = _dot(_bf(_row_repeat(spt[g], fexp)), gblkt_q) * mfo_q
        outs.append(_dot(y2b[g], _bf(q)))
    out = jnp.concatenate(outs, axis=0).reshape(nb, tl, d2)
    return out + bias_full[None]


def _chain_multi(x, ps):
    l_s, l_r = ps[0:8], ps[8:16]
    lng, lnb = ps[16], ps[17]
    l_e = ps[18:26]
    y = _gc_multi(x, *l_s, True)
    y = jnp.concatenate([y, y], axis=1)
    z = _gc_multi(y, *l_r, False)
    z = jnp.tanh(_ln_multi(z, lng, lnb))
    y = z + y
    y = y[:, :_T, :]
    y = _gc_multi(y, *l_e, False)
    return y + x


def _conv3_multi(x_bf, w0, w1, w2, b):
    nb, tl, c = x_bf.shape
    xm = jnp.concatenate([x_bf[:, :1], x_bf[:, :-1]], axis=1)
    xp = jnp.concatenate([x_bf[:, 1:], x_bf[:, -1:]], axis=1)
    r = (_dot(xm.reshape(nb * tl, c), w0)
         + _dot(x_bf.reshape(nb * tl, c), w1)
         + _dot(xp.reshape(nb * tl, c), w2) + b)
    return r.reshape(nb, tl, r.shape[1])


def _gaze_multi(x, gz):
    (w10, w11, w12, b1, g1, be1,
     w20, w21, w22, b2, g2, be2,
     w30, w31, w32, b3, g3, be3,
     w40, w41, w42, b4) = gz
    y = jnp.tanh(_ln_multi(_conv3_multi(_bf(x), w10, w11, w12, b1), g1, be1))
    y = jnp.tanh(_ln_multi(_conv3_multi(_bf(y), w20, w21, w22, b2), g2, be2))
    y = jnp.tanh(_ln_multi(_conv3_multi(_bf(y), w30, w31, w32, b3), g3, be3))
    y = jnp.tanh(_conv3_multi(_bf(y), w40, w41, w42, b4))
    return y


def _net_kernel(*refs):
    pose_ref, gaze_ref = refs[0], refs[1]
    consts = [r[...] for r in refs[2:-1]]
    o_ref = refs[-1]
    gz = consts[0:22]
    po = consts[22:48]
    mn = consts[48:74]
    xp = pose_ref[...].astype(jnp.float32)
    xg = gaze_ref[...].astype(jnp.float32)
    gout = _gaze_multi(xg, gz)
    pout = _chain_multi(xp, po)
    xm = jnp.concatenate([pout, gout], axis=2)
    o_ref[...] = _chain_multi(xm, mn).astype(o_ref.dtype)


def _gblk(node_n, f):
    return jnp.kron(jnp.eye(node_n, dtype=jnp.float32),
                    jnp.ones((f, 1), jnp.float32))


def _prep_layer(att_t, fw, att_s, bias, node_n, start):
    fin, fout = fw.shape
    at = att_t[:, :, 0]
    d1 = at.shape[1] // 2
    wt = _bf(jnp.concatenate([at[:, :d1], at[:, d1:]], axis=0))
    wblk = _bf(jnp.kron(jnp.eye(node_n, dtype=fw.dtype), fw))
    asp = att_s[:, :, 0]
    tl = asp.shape[1] // (2 * fout)
    wsl = asp[:, :tl * fout].reshape(_H, tl, fout)
    wsr = asp[:, tl * fout:].reshape(_H, tl, fout)
    ws = jnp.concatenate([wsl, wsr], axis=0)
    if start:
        wsr_ = jnp.einsum("htf,gf->htg", ws, fw)
        frr = fin
    else:
        wsr_ = ws
        frr = fout
    wsp = _bf(jnp.transpose(wsr_, (0, 2, 1)).reshape(2 * _H * frr, tl))
    fmask = jnp.tile(jnp.eye(frr, dtype=jnp.float32), (2 * _H, node_n))
    red_gblk = _bf(_gblk(node_n, frr))
    bias_full = jnp.broadcast_to(bias[:, None].astype(jnp.float32),
                                 (tl, node_n * fout)) + jnp.zeros(
                                     (tl, node_n * fout), jnp.float32)
    gblkt_q = _bf(jnp.transpose(_gblk(node_n, fout)))
    mfo_q = jnp.tile(jnp.eye(fout, dtype=jnp.float32),
                     (node_n, node_n)) * (1.0 / _H)
    return [wt, wblk, wsp, fmask, red_gblk, bias_full, gblkt_q, mfo_q]


def _taps(w, b):
    return [_bf(jnp.transpose(w[:, :, 0])), _bf(jnp.transpose(w[:, :, 1])),
            _bf(jnp.transpose(w[:, :, 2])), b[None, :].astype(jnp.float32)]


def kernel(x,
           gz_w1, gz_b1, gz_ln1_g, gz_ln1_b,
           gz_w2, gz_b2, gz_ln2_g, gz_ln2_b,
           gz_w3, gz_b3, gz_ln3_g, gz_ln3_b,
           gz_w4, gz_b4,
           po_start_att_t, po_start_fw, po_start_att_s, po_start_bias,
           po_res0_att_t, po_res0_fw, po_res0_att_s, po_res0_bias,
           po_res0_lng, po_res0_lnb,
           po_end_att_t, po_end_fw, po_end_att_s, po_end_bias,
           mn_start_att_t, mn_start_fw, mn_start_att_s, mn_start_bias,
           mn_res0_att_t, mn_res0_fw, mn_res0_att_s, mn_res0_bias,
           mn_res0_lng, mn_res0_lnb,
           mn_end_att_t, mn_end_fw, mn_end_att_s, mn_end_bias):
    B, fin, node_n, T = x.shape
    npo = node_n - 1
    pose_l = jnp.transpose(x[:, :, :-1, :], (0, 3, 2, 1)).reshape(B, T, npo * fin)
    gaze_l = jnp.transpose(x[:, :, -1, :], (0, 2, 1))

    gz = (_taps(gz_w1, gz_b1) + [jnp.transpose(gz_ln1_g), jnp.transpose(gz_ln1_b)]
          + _taps(gz_w2, gz_b2) + [jnp.transpose(gz_ln2_g), jnp.transpose(gz_ln2_b)]
          + _taps(gz_w3, gz_b3) + [jnp.transpose(gz_ln3_g), jnp.transpose(gz_ln3_b)]
          + _taps(gz_w4, gz_b4))

    def chain_consts(s_att_t, s_fw, s_att_s, s_b, r_att_t, r_fw, r_att_s, r_b,
                     lng, lnb, e_att_t, e_fw, e_att_s, e_b, n):
        latent = s_fw.shape[1]
        dl = n * latent
        cs = _prep_layer(s_att_t, s_fw, s_att_s, s_b, n, True)
        cs += _prep_layer(r_att_t, r_fw, r_att_s, r_b, n, False)
        cs += [jnp.transpose(lng, (2, 1, 0)).reshape(2 * _T, dl),
               jnp.transpose(lnb, (2, 1, 0)).reshape(2 * _T, dl)]
        cs += _prep_layer(e_att_t, e_fw, e_att_s, e_b, n, False)
        return cs

    po = chain_consts(po_start_att_t, po_start_fw, po_start_att_s, po_start_bias,
                      po_res0_att_t, po_res0_fw, po_res0_att_s, po_res0_bias,
                      po_res0_lng, po_res0_lnb,
                      po_end_att_t, po_end_fw, po_end_att_s, po_end_bias, npo)
    mn = chain_consts(mn_start_att_t, mn_start_fw, mn_start_att_s, mn_start_bias,
                      mn_res0_att_t, mn_res0_fw, mn_res0_att_s, mn_res0_bias,
                      mn_res0_lng, mn_res0_lnb,
                      mn_end_att_t, mn_end_fw, mn_end_att_s, mn_end_bias, node_n)

    consts = gz + po + mn

    def _const_spec(a):
        if a.ndim == 3:
            return pl.BlockSpec(a.shape, lambda b: (0, 0, 0))
        return pl.BlockSpec(a.shape, lambda b: (0, 0))

    out = pl.pallas_call(
        _net_kernel,
        out_shape=jax.ShapeDtypeStruct((B, T, node_n * fin), jnp.float32),
        grid=(B // _BB,),
        in_specs=[pl.BlockSpec((_BB, T, npo * fin), lambda b: (b, 0, 0)),
                  pl.BlockSpec((_BB, T, fin), lambda b: (b, 0, 0))]
                 + [_const_spec(a) for a in consts],
        out_specs=pl.BlockSpec((_BB, T, node_n * fin), lambda b: (b, 0, 0)),
        compiler_params=pltpu.CompilerParams(dimension_semantics=("parallel",)),
    )(pose_l, gaze_l, *consts)

    return jnp.transpose(out.reshape(B, T, node_n, fin), (0, 3, 2, 1))


# speedup vs baseline: 1.5603x; 1.3138x over previous
import functools

import jax
import jax.numpy as jnp
from jax import lax
from jax.experimental import pallas as pl
from jax.experimental.pallas import tpu as pltpu

_BB = 8
_T = 64
_H = 4


def _bf(x):
    return x.astype(jnp.bfloat16)


def _dot(a, b):
    return jnp.dot(a, b, preferred_element_type=jnp.float32)


def _leaky(x):
    return jnp.maximum(x, 0.2 * x)


def _row_repeat(s, f):
    n = s.shape[0]
    return jnp.broadcast_to(s[:, None, :], (n, f, n)).reshape(n * f, n)


def _ln_multi(y, gamma, beta, eps=1e-5):
    n_el = float(y.shape[1] * y.shape[2])
    s1 = jnp.sum(jnp.sum(y, axis=2, keepdims=True), axis=1, keepdims=True)
    s2 = jnp.sum(jnp.sum(y * y, axis=2, keepdims=True), axis=1, keepdims=True)
    mean = s1 / n_el
    var = s2 / n_el - mean * mean
    return (y - mean) * lax.rsqrt(var + eps) * gamma[None] + beta[None]


def _gc_multi(ys, wt, wblk, wsp, fmask, red_gblk, bias_full, gblkt_q, mfo_q,
              red_from_a1):
    nb, tl, d1 = ys.shape
    d2 = wblk.shape[1]
    n = red_gblk.shape[1]
    fr = fmask.shape[0] // (2 * _H)
    dr = fmask.shape[1]
    fexp = d2 // n
    inv_h = 1.0 / _H
    ys_bf = _bf(ys)

    projs = [lax.dot_general(wt, ys_bf[g], (((1,), (1,)), ((), ())),
                             preferred_element_type=jnp.float32)
             for g in range(nb)]
    proj = jnp.concatenate(projs, axis=0).reshape(nb, 2 * _H, tl)
    pl_, pr_ = proj[:, :_H], proj[:, _H:]
    emax = _leaky(pl_ + jnp.max(pr_, axis=-1, keepdims=True))
    e = _leaky(pr_[:, :, :, None] + pl_[:, :, None, :]) - emax[:, :, None, :]
    p = jnp.exp(e)
    p = p / (jnp.sum(p, axis=2, keepdims=True) * float(_H))
    sp = jnp.sum(p, axis=1)

    a1b = [_bf(lax.dot_general(_bf(sp[g]), ys_bf[g], (((0,), (0,)), ((), ())),
                               preferred_element_type=jnp.float32))
           for g in range(nb)]
    y2_flat = _dot(jnp.concatenate(a1b, axis=0), wblk)
    y2 = y2_flat.reshape(nb, tl, d2)
    y2b = _bf(y2)

    red_src = a1b if red_from_a1 else [y2b[g] for g in range(nb)]
    ss = [jnp.sum((_dot(wsp, src) * fmask).reshape(2 * _H, fr, dr), axis=1)
          for src in red_src]
    proj_s = _dot(_bf(jnp.concatenate(ss, axis=0)), red_gblk)
    ps3 = proj_s.reshape(nb, 2 * _H, n)
    r_, l_ = ps3[:, _H:], ps3[:, :_H]
    etmax = _leaky(l_ + jnp.max(r_, axis=-1, keepdims=True))
    et = _leaky(r_[:, :, :, None] + l_[:, :, None, :]) - etmax[:, :, None, :]
    pt = jnp.exp(et)
    pt = pt / jnp.sum(pt, axis=2, keepdims=True)
    spt = jnp.sum(pt, axis=1)

    outs = []
    for g in range(nb):
        q = _dot(_bf(_row_repeat(spt[g], fexp)), gblkt_q) * mfo_q
        outs.append(_dot(y2b[g], _bf(q)))
    out = jnp.concatenate(outs, axis=0).reshape(nb, tl, d2)
    return out + bias_full[None]


def _chain_multi(x, ps):
    l_s, l_r = ps[0:8], ps[8:16]
    lng, lnb = ps[16], ps[17]
    l_e = ps[18:26]
    y = _gc_multi(x, *l_s, True)
    y = jnp.concatenate([y, y], axis=1)
    z = _gc_multi(y, *l_r, False)
    z = jnp.tanh(_ln_multi(z, lng, lnb))
    y = z + y
    y = y[:, :_T, :]
    y = _gc_multi(y, *l_e, False)
    return y + x


def _conv3_multi(x_bf, w0, w1, w2, b):
    nb, tl, c = x_bf.shape
    xm = jnp.concatenate([x_bf[:, :1], x_bf[:, :-1]], axis=1)
    xp = jnp.concatenate([x_bf[:, 1:], x_bf[:, -1:]], axis=1)
    r = (_dot(xm.reshape(nb * tl, c), w0)
         + _dot(x_bf.reshape(nb * tl, c), w1)
         + _dot(xp.reshape(nb * tl, c), w2) + b)
    return r.reshape(nb, tl, r.shape[1])


def _gaze_multi(x, gz):
    (w10, w11, w12, b1, g1, be1,
     w20, w21, w22, b2, g2, be2,
     w30, w31, w32, b3, g3, be3,
     w40, w41, w42, b4) = gz
    y = jnp.tanh(_ln_multi(_conv3_multi(_bf(x), w10, w11, w12, b1), g1, be1))
    y = jnp.tanh(_ln_multi(_conv3_multi(_bf(y), w20, w21, w22, b2), g2, be2))
    y = jnp.tanh(_ln_multi(_conv3_multi(_bf(y), w30, w31, w32, b3), g3, be3))
    y = jnp.tanh(_conv3_multi(_bf(y), w40, w41, w42, b4))
    return y


def _net_kernel(*refs):
    pose_ref, gaze_ref = refs[0], refs[1]
    consts = [r[...] for r in refs[2:-1]]
    o_ref = refs[-1]
    gz = consts[0:22]
    po = consts[22:48]
    mn = consts[48:74]
    xp = pose_ref[...].astype(jnp.float32)
    xg = gaze_ref[...].astype(jnp.float32)
    gout = _gaze_multi(xg, gz)
    pout = _chain_multi(xp, po)
    xm = jnp.concatenate([pout, gout], axis=2)
    o_ref[...] = _chain_multi(xm, mn).astype(o_ref.dtype)


def _gblk(node_n, f):
    return jnp.kron(jnp.eye(node_n, dtype=jnp.float32),
                    jnp.ones((f, 1), jnp.float32))


def _prep_layer(att_t, fw, att_s, bias, node_n, start):
    fin, fout = fw.shape
    at = att_t[:, :, 0]
    d1 = at.shape[1] // 2
    wt = _bf(jnp.concatenate([at[:, :d1], at[:, d1:]], axis=0))
    wblk = _bf(jnp.kron(jnp.eye(node_n, dtype=fw.dtype), fw))
    asp = att_s[:, :, 0]
    tl = asp.shape[1] // (2 * fout)
    wsl = asp[:, :tl * fout].reshape(_H, tl, fout)
    wsr = asp[:, tl * fout:].reshape(_H, tl, fout)
    ws = jnp.concatenate([wsl, wsr], axis=0)
    if start:
        wsr_ = jnp.einsum("htf,gf->htg", ws, fw)
        frr = fin
    else:
        wsr_ = ws
        frr = fout
    wsp = _bf(jnp.transpose(wsr_, (0, 2, 1)).reshape(2 * _H * frr, tl))
    fmask = jnp.tile(jnp.eye(frr, dtype=jnp.float32), (2 * _H, node_n))
    red_gblk = _bf(_gblk(node_n, frr))
    bias_full = jnp.broadcast_to(bias[:, None].astype(jnp.float32),
                                 (tl, node_n * fout)) + jnp.zeros(
                                     (tl, node_n * fout), jnp.float32)
    gblkt_q = _bf(jnp.transpose(_gblk(node_n, fout)))
    mfo_q = jnp.tile(jnp.eye(fout, dtype=jnp.float32),
                     (node_n, node_n)) * (1.0 / _H)
    return [wt, wblk, wsp, fmask, red_gblk, bias_full, gblkt_q, mfo_q]


def _taps(w, b):
    return [_bf(jnp.transpose(w[:, :, 0])), _bf(jnp.transpose(w[:, :, 1])),
            _bf(jnp.transpose(w[:, :, 2])), b[None, :].astype(jnp.float32)]


def kernel(x,
           gz_w1, gz_b1, gz_ln1_g, gz_ln1_b,
           gz_w2, gz_b2, gz_ln2_g, gz_ln2_b,
           gz_w3, gz_b3, gz_ln3_g, gz_ln3_b,
           gz_w4, gz_b4,
           po_start_att_t, po_start_fw, po_start_att_s, po_start_bias,
           po_res0_att_t, po_res0_fw, po_res0_att_s, po_res0_bias,
           po_res0_lng, po_res0_lnb,
           po_end_att_t, po_end_fw, po_end_att_s, po_end_bias,
           mn_start_att_t, mn_start_fw, mn_start_att_s, mn_start_bias,
           mn_res0_att_t, mn_res0_fw, mn_res0_att_s, mn_res0_bias,
           mn_res0_lng, mn_res0_lnb,
           mn_end_att_t, mn_end_fw, mn_end_att_s, mn_end_bias):
    B, fin, node_n, T = x.shape
    npo = node_n - 1
    pose_l = jnp.transpose(x[:, :, :-1, :], (0, 3, 2, 1)).reshape(B, T, npo * fin)
    gaze_l = jnp.transpose(x[:, :, -1, :], (0, 2, 1))

    gz = (_taps(gz_w1, gz_b1) + [jnp.transpose(gz_ln1_g), jnp.transpose(gz_ln1_b)]
          + _taps(gz_w2, gz_b2) + [jnp.transpose(gz_ln2_g), jnp.transpose(gz_ln2_b)]
          + _taps(gz_w3, gz_b3) + [jnp.transpose(gz_ln3_g), jnp.transpose(gz_ln3_b)]
          + _taps(gz_w4, gz_b4))

    def chain_consts(s_att_t, s_fw, s_att_s, s_b, r_att_t, r_fw, r_att_s, r_b,
                     lng, lnb, e_att_t, e_fw, e_att_s, e_b, n):
        latent = s_fw.shape[1]
        dl = n * latent
        cs = _prep_layer(s_att_t, s_fw, s_att_s, s_b, n, True)
        cs += _prep_layer(r_att_t, r_fw, r_att_s, r_b, n, False)
        cs += [jnp.transpose(lng, (2, 1, 0)).reshape(2 * _T, dl),
               jnp.transpose(lnb, (2, 1, 0)).reshape(2 * _T, dl)]
        cs += _prep_layer(e_att_t, e_fw, e_att_s, e_b, n, False)
        return cs

    po = chain_consts(po_start_att_t, po_start_fw, po_start_att_s, po_start_bias,
                      po_res0_att_t, po_res0_fw, po_res0_att_s, po_res0_bias,
                      po_res0_lng, po_res0_lnb,
                      po_end_att_t, po_end_fw, po_end_att_s, po_end_bias, npo)
    mn = chain_consts(mn_start_att_t, mn_start_fw, mn_start_att_s, mn_start_bias,
                      mn_res0_att_t, mn_res0_fw, mn_res0_att_s, mn_res0_bias,
                      mn_res0_lng, mn_res0_lnb,
                      mn_end_att_t, mn_end_fw, mn_end_att_s, mn_end_bias, node_n)

    consts = gz + po + mn

    def _const_spec(a):
        if a.ndim == 3:
            return pl.BlockSpec(a.shape, lambda b: (0, 0, 0))
        return pl.BlockSpec(a.shape, lambda b: (0, 0))

    out = pl.pallas_call(
        _net_kernel,
        out_shape=jax.ShapeDtypeStruct((B, T, node_n * fin), jnp.float32),
        grid=(B // _BB,),
        in_specs=[pl.BlockSpec((_BB, T, npo * fin), lambda b: (b, 0, 0)),
                  pl.BlockSpec((_BB, T, fin), lambda b: (b, 0, 0))]
                 + [_const_spec(a) for a in consts],
        out_specs=pl.BlockSpec((_BB, T, node_n * fin), lambda b: (b, 0, 0)),
        compiler_params=pltpu.CompilerParams(dimension_semantics=("parallel",)),
    )(pose_l, gaze_l, *consts)

    return jnp.transpose(out.reshape(B, T, node_n, fin), (0, 3, 2, 1))


# speedup vs baseline: 1.7377x; 1.1136x over previous
import functools

import jax
import jax.numpy as jnp
from jax import lax
from jax.experimental import pallas as pl
from jax.experimental.pallas import tpu as pltpu

_BB = 16
_T = 64
_H = 4


def _bf(x):
    return x.astype(jnp.bfloat16)


def _dot(a, b):
    return jnp.dot(a, b, preferred_element_type=jnp.float32)


def _leaky(x):
    return jnp.maximum(x, 0.2 * x)


def _row_repeat(s, f):
    n = s.shape[0]
    return jnp.broadcast_to(s[:, None, :], (n, f, n)).reshape(n * f, n)


def _ln_multi(y, gamma, beta, eps=1e-5):
    n_el = float(y.shape[1] * y.shape[2])
    s1 = jnp.sum(jnp.sum(y, axis=2, keepdims=True), axis=1, keepdims=True)
    s2 = jnp.sum(jnp.sum(y * y, axis=2, keepdims=True), axis=1, keepdims=True)
    mean = s1 / n_el
    var = s2 / n_el - mean * mean
    return (y - mean) * lax.rsqrt(var + eps) * gamma[None] + beta[None]


def _gc_multi(ys, wt, wblk, wsp, fmask, red_gblk, bias_full, gblkt_q, mfo_q,
              red_from_a1):
    nb, tl, d1 = ys.shape
    d2 = wblk.shape[1]
    n = red_gblk.shape[1]
    fr = fmask.shape[0] // (2 * _H)
    dr = fmask.shape[1]
    fexp = d2 // n
    inv_h = 1.0 / _H
    ys_bf = _bf(ys)

    projs = [lax.dot_general(wt, ys_bf[g], (((1,), (1,)), ((), ())),
                             preferred_element_type=jnp.float32)
             for g in range(nb)]
    proj = jnp.concatenate(projs, axis=0).reshape(nb, 2 * _H, tl)
    pl_, pr_ = proj[:, :_H], proj[:, _H:]
    emax = _leaky(pl_ + jnp.max(pr_, axis=-1, keepdims=True))
    e = _leaky(pr_[:, :, :, None] + pl_[:, :, None, :]) - emax[:, :, None, :]
    p = jnp.exp(e)
    p = p / (jnp.sum(p, axis=2, keepdims=True) * float(_H))
    sp = jnp.sum(p, axis=1)

    a1b = [_bf(lax.dot_general(_bf(sp[g]), ys_bf[g], (((0,), (0,)), ((), ())),
                               preferred_element_type=jnp.float32))
           for g in range(nb)]
    y2_flat = _dot(jnp.concatenate(a1b, axis=0), wblk)
    y2 = y2_flat.reshape(nb, tl, d2)
    y2b = _bf(y2)

    red_src = a1b if red_from_a1 else [y2b[g] for g in range(nb)]
    ss = [jnp.sum((_dot(wsp, src) * fmask).reshape(2 * _H, fr, dr), axis=1)
          for src in red_src]
    proj_s = _dot(_bf(jnp.concatenate(ss, axis=0)), red_gblk)
    ps3 = proj_s.reshape(nb, 2 * _H, n)
    r_, l_ = ps3[:, _H:], ps3[:, :_H]
    etmax = _leaky(l_ + jnp.max(r_, axis=-1, keepdims=True))
    et = _leaky(r_[:, :, :, None] + l_[:, :, None, :]) - etmax[:, :, None, :]
    pt = jnp.exp(et)
    pt = pt / jnp.sum(pt, axis=2, keepdims=True)
    spt = jnp.sum(pt, axis=1)

    outs = []
    for g in range(nb):
        q = _dot(_bf(_row_repeat(spt[g], fexp)), gblkt_q) * mfo_q
        outs.append(_dot(y2b[g], _bf(q)))
    out = jnp.concatenate(outs, axis=0).reshape(nb, tl, d2)
    return out + bias_full[None]


def _chain_multi(x, ps):
    l_s, l_r = ps[0:8], ps[8:16]
    lng, lnb = ps[16], ps[17]
    l_e = ps[18:26]
    y = _gc_multi(x, *l_s, True)
    y = jnp.concatenate([y, y], axis=1)
    z = _gc_multi(y, *l_r, False)
    z = jnp.tanh(_ln_multi(z, lng, lnb))
    y = z + y
    y = y[:, :_T, :]
    y = _gc_multi(y, *l_e, False)
    return y + x


def _conv3_multi(x_bf, w0, w1, w2, b):
    nb, tl, c = x_bf.shape
    xm = jnp.concatenate([x_bf[:, :1], x_bf[:, :-1]], axis=1)
    xp = jnp.concatenate([x_bf[:, 1:], x_bf[:, -1:]], axis=1)
    r = (_dot(xm.reshape(nb * tl, c), w0)
         + _dot(x_bf.reshape(nb * tl, c), w1)
         + _dot(xp.reshape(nb * tl, c), w2) + b)
    return r.reshape(nb, tl, r.shape[1])


def _gaze_multi(x, gz):
    (w10, w11, w12, b1, g1, be1,
     w20, w21, w22, b2, g2, be2,
     w30, w31, w32, b3, g3, be3,
     w40, w41, w42, b4) = gz
    y = jnp.tanh(_ln_multi(_conv3_multi(_bf(x), w10, w11, w12, b1), g1, be1))
    y = jnp.tanh(_ln_multi(_conv3_multi(_bf(y), w20, w21, w22, b2), g2, be2))
    y = jnp.tanh(_ln_multi(_conv3_multi(_bf(y), w30, w31, w32, b3), g3, be3))
    y = jnp.tanh(_conv3_multi(_bf(y), w40, w41, w42, b4))
    return y


def _net_kernel(*refs):
    pose_ref, gaze_ref = refs[0], refs[1]
    consts = [r[...] for r in refs[2:-1]]
    o_ref = refs[-1]
    gz = consts[0:22]
    po = consts[22:48]
    mn = consts[48:74]
    xp = pose_ref[...].astype(jnp.float32)
    xg = gaze_ref[...].astype(jnp.float32)
    gout = _gaze_multi(xg, gz)
    pout = _chain_multi(xp, po)
    xm = jnp.concatenate([pout, gout], axis=2)
    o_ref[...] = _chain_multi(xm, mn).astype(o_ref.dtype)


def _gblk(node_n, f):
    return jnp.kron(jnp.eye(node_n, dtype=jnp.float32),
                    jnp.ones((f, 1), jnp.float32))


def _prep_layer(att_t, fw, att_s, bias, node_n, start):
    fin, fout = fw.shape
    at = att_t[:, :, 0]
    d1 = at.shape[1] // 2
    wt = _bf(jnp.concatenate([at[:, :d1], at[:, d1:]], axis=0))
    wblk = _bf(jnp.kron(jnp.eye(node_n, dtype=fw.dtype), fw))
    asp = att_s[:, :, 0]
    tl = asp.shape[1] // (2 * fout)
    wsl = asp[:, :tl * fout].reshape(_H, tl, fout)
    wsr = asp[:, tl * fout:].reshape(_H, tl, fout)
    ws = jnp.concatenate([wsl, wsr], axis=0)
    if start:
        wsr_ = jnp.einsum("htf,gf->htg", ws, fw)
        frr = fin
    else:
        wsr_ = ws
        frr = fout
    wsp = _bf(jnp.transpose(wsr_, (0, 2, 1)).reshape(2 * _H * frr, tl))
    fmask = jnp.tile(jnp.eye(frr, dtype=jnp.float32), (2 * _H, node_n))
    red_gblk = _bf(_gblk(node_n, frr))
    bias_full = jnp.broadcast_to(bias[:, None].astype(jnp.float32),
                                 (tl, node_n * fout)) + jnp.zeros(
                                     (tl, node_n * fout), jnp.float32)
    gblkt_q = _bf(jnp.transpose(_gblk(node_n, fout)))
    mfo_q = jnp.tile(jnp.eye(fout, dtype=jnp.float32),
                     (node_n, node_n)) * (1.0 / _H)
    return [wt, wblk, wsp, fmask, red_gblk, bias_full, gblkt_q, mfo_q]


def _taps(w, b):
    return [_bf(jnp.transpose(w[:, :, 0])), _bf(jnp.transpose(w[:, :, 1])),
            _bf(jnp.transpose(w[:, :, 2])), b[None, :].astype(jnp.float32)]


def kernel(x,
           gz_w1, gz_b1, gz_ln1_g, gz_ln1_b,
           gz_w2, gz_b2, gz_ln2_g, gz_ln2_b,
           gz_w3, gz_b3, gz_ln3_g, gz_ln3_b,
           gz_w4, gz_b4,
           po_start_att_t, po_start_fw, po_start_att_s, po_start_bias,
           po_res0_att_t, po_res0_fw, po_res0_att_s, po_res0_bias,
           po_res0_lng, po_res0_lnb,
           po_end_att_t, po_end_fw, po_end_att_s, po_end_bias,
           mn_start_att_t, mn_start_fw, mn_start_att_s, mn_start_bias,
           mn_res0_att_t, mn_res0_fw, mn_res0_att_s, mn_res0_bias,
           mn_res0_lng, mn_res0_lnb,
           mn_end_att_t, mn_end_fw, mn_end_att_s, mn_end_bias):
    B, fin, node_n, T = x.shape
    npo = node_n - 1
    pose_l = jnp.transpose(x[:, :, :-1, :], (0, 3, 2, 1)).reshape(B, T, npo * fin)
    gaze_l = jnp.transpose(x[:, :, -1, :], (0, 2, 1))

    gz = (_taps(gz_w1, gz_b1) + [jnp.transpose(gz_ln1_g), jnp.transpose(gz_ln1_b)]
          + _taps(gz_w2, gz_b2) + [jnp.transpose(gz_ln2_g), jnp.transpose(gz_ln2_b)]
          + _taps(gz_w3, gz_b3) + [jnp.transpose(gz_ln3_g), jnp.transpose(gz_ln3_b)]
          + _taps(gz_w4, gz_b4))

    def chain_consts(s_att_t, s_fw, s_att_s, s_b, r_att_t, r_fw, r_att_s, r_b,
                     lng, lnb, e_att_t, e_fw, e_att_s, e_b, n):
        latent = s_fw.shape[1]
        dl = n * latent
        cs = _prep_layer(s_att_t, s_fw, s_att_s, s_b, n, True)
        cs += _prep_layer(r_att_t, r_fw, r_att_s, r_b, n, False)
        cs += [jnp.transpose(lng, (2, 1, 0)).reshape(2 * _T, dl),
               jnp.transpose(lnb, (2, 1, 0)).reshape(2 * _T, dl)]
        cs += _prep_layer(e_att_t, e_fw, e_att_s, e_b, n, False)
        return cs

    po = chain_consts(po_start_att_t, po_start_fw, po_start_att_s, po_start_bias,
                      po_res0_att_t, po_res0_fw, po_res0_att_s, po_res0_bias,
                      po_res0_lng, po_res0_lnb,
                      po_end_att_t, po_end_fw, po_end_att_s, po_end_bias, npo)
    mn = chain_consts(mn_start_att_t, mn_start_fw, mn_start_att_s, mn_start_bias,
                      mn_res0_att_t, mn_res0_fw, mn_res0_att_s, mn_res0_bias,
                      mn_res0_lng, mn_res0_lnb,
                      mn_end_att_t, mn_end_fw, mn_end_att_s, mn_end_bias, node_n)

    consts = gz + po + mn

    def _const_spec(a):
        if a.ndim == 3:
            return pl.BlockSpec(a.shape, lambda b: (0, 0, 0))
        return pl.BlockSpec(a.shape, lambda b: (0, 0))

    out = pl.pallas_call(
        _net_kernel,
        out_shape=jax.ShapeDtypeStruct((B, T, node_n * fin), jnp.float32),
        grid=(B // _BB,),
        in_specs=[pl.BlockSpec((_BB, T, npo * fin), lambda b: (b, 0, 0)),
                  pl.BlockSpec((_BB, T, fin), lambda b: (b, 0, 0))]
                 + [_const_spec(a) for a in consts],
        out_specs=pl.BlockSpec((_BB, T, node_n * fin), lambda b: (b, 0, 0)),
        compiler_params=pltpu.CompilerParams(dimension_semantics=("parallel",)),
    )(pose_l, gaze_l, *consts)

    return jnp.transpose(out.reshape(B, T, node_n, fin), (0, 3, 2, 1))
